# baseline (speedup 1.0000x reference)
import functools

import jax
import jax.numpy as jnp
from jax import lax
from jax.experimental import pallas as pl
from jax.experimental.pallas import tpu as pltpu
from jax.experimental.pallas import tpu_sc as plsc

T = 2048
C = 768
H = 12
HD = 64
NMEM = 8192
TOPK = 3
TB = 256
NT = T // TB

def _qkv_body(x_ref, w_ref, qkv_ref, kv_ref):
    qkv = jnp.dot(x_ref[...], w_ref[...], preferred_element_type=jnp.float32)
    qkv_ref[...] = qkv
    kv_ref[:, 0, :] = qkv[:, C:2 * C]
    kv_ref[:, 1, :] = qkv[:, 2 * C:3 * C]


def _qkv_call(x2, c_attn_w):
    return pl.pallas_call(
        _qkv_body,
        grid=(NT,),
        in_specs=[
            pl.BlockSpec((TB, C), lambda t: (t, 0)),
            pl.BlockSpec((C, 3 * C), lambda t: (0, 0)),
        ],
        out_specs=[
            pl.BlockSpec((TB, 3 * C), lambda t: (t, 0)),
            pl.BlockSpec((TB, 2, C), lambda t: (t, 0, 0)),
        ],
        out_shape=[
            jax.ShapeDtypeStruct((T, 3 * C), jnp.float32),
            jax.ShapeDtypeStruct((T, 2, C), jnp.float32),
        ],
    )(x2, c_attn_w)


def _attn_body(q_ref, k_ref, v_ref, y_ref):
    tb = pl.program_id(1)
    row = lax.broadcasted_iota(jnp.int32, (TB, TB), 0)
    col = lax.broadcasted_iota(jnp.int32, (TB, TB), 1)
    for h in range(2):
        q = q_ref[:, h * HD:(h + 1) * HD]

        def body(c, carry):
            m, den, acc = carry
            k = k_ref[pl.ds(c * TB, TB), h * HD:(h + 1) * HD]
            v = v_ref[pl.ds(c * TB, TB), h * HD:(h + 1) * HD]
            s = lax.dot_general(q, k, (((1,), (1,)), ((), ())),
                                preferred_element_type=jnp.float32)
            s = s * 0.125
            s = jnp.where(jnp.logical_and(c == tb, col > row),
                          jnp.float32(-1e30), s)
            m_new = jnp.maximum(m, jnp.max(s, axis=1, keepdims=True))
            alpha = jnp.exp(m - m_new)
            p = jnp.exp(s - m_new)
            den = den * alpha + jnp.sum(p, axis=1, keepdims=True)
            acc = acc * alpha + jnp.dot(p, v, preferred_element_type=jnp.float32)
            return m_new, den, acc

        m0 = jnp.full((TB, 1), jnp.float32(-3e38))
        den0 = jnp.zeros((TB, 1), jnp.float32)
        acc0 = jnp.zeros((TB, HD), jnp.float32)
        m, den, acc = lax.fori_loop(0, tb + 1, body, (m0, den0, acc0))
        y_ref[:, h * HD:(h + 1) * HD] = acc / den


def _attn_call(qkv):
    return pl.pallas_call(
        _attn_body,
        grid=(H // 2, NT),
        in_specs=[
            pl.BlockSpec((TB, 128), lambda hp, t: (t, hp)),
            pl.BlockSpec((T, 128), lambda hp, t: (0, 6 + hp)),
            pl.BlockSpec((T, 128), lambda hp, t: (0, 12 + hp)),
        ],
        out_specs=pl.BlockSpec((TB, 128), lambda hp, t: (t, hp)),
        out_shape=jax.ShapeDtypeStruct((T, C), jnp.float32),
    )(qkv, qkv, qkv)


def _topk_body(q_ref, mk_ref, idx_ref):
    s = lax.dot_general(q_ref[...], mk_ref[...], (((1,), (1,)), ((), ())),
                        preferred_element_type=jnp.float32)
    col = lax.broadcasted_iota(jnp.int32, (TB, NMEM), 1)
    picks = []
    for _ in range(TOPK):
        m = jnp.max(s, axis=1, keepdims=True)
        i = jnp.min(jnp.where(s == m, col, NMEM), axis=1, keepdims=True)
        picks.append(i)
        s = jnp.where(col == i, jnp.float32(-3e38), s)
    lane = lax.broadcasted_iota(jnp.int32, (TB, 128), 1)
    out = jnp.where(lane == 0, picks[0],
                    jnp.where(lane == 1, picks[1],
                              jnp.where(lane == 2, picks[2], 0)))
    idx_ref[...] = out


def _topk_call(qkv, table):
    return pl.pallas_call(
        _topk_body,
        grid=(NT,),
        in_specs=[
            pl.BlockSpec((TB, C), lambda t: (t, 0)),
            pl.BlockSpec((NMEM, C), lambda t: (0, 0)),
        ],
        out_specs=pl.BlockSpec((TB, 128), lambda t: (t, 0)),
        out_shape=jax.ShapeDtypeStruct((T, 128), jnp.int32),
    )(qkv, table)


_NROWS = TOPK * T
_NW = 32
_RPW = _NROWS // _NW
_CHUNK = 48


def _gather_call(table, idx_flat):
    mesh = plsc.VectorSubcoreMesh(core_axis_name="c", subcore_axis_name="s")

    @functools.partial(
        pl.kernel,
        mesh=mesh,
        out_type=jax.ShapeDtypeStruct((_NROWS, 2 * C), jnp.float32),
        scratch_types=[
            pltpu.VMEM((_RPW,), jnp.int32),
            pltpu.VMEM((_CHUNK, 2 * C), jnp.float32),
            pltpu.SemaphoreType.DMA,
        ],
    )
    def _gather(table_hbm, idx_hbm, out_hbm, idx_v, rows_v, sem):
        wid = lax.axis_index("s") * 2 + lax.axis_index("c")
        base = wid * _RPW
        pltpu.sync_copy(idx_hbm.at[pl.ds(base, _RPW)], idx_v)
        for ch in range(_RPW // _CHUNK):
            pltpu.async_copy(
                table_hbm.at[idx_v.at[pl.ds(ch * _CHUNK, _CHUNK)]], rows_v, sem
            ).wait()
            pltpu.sync_copy(rows_v, out_hbm.at[pl.ds(base + ch * _CHUNK, _CHUNK)])

    return _gather(table, idx_flat)


def _mem_body(q_ref, g_ref, y_ref, gate_ref, jmat_ref, w_ref, out_ref):
    q = q_ref[...]
    qk = []
    for kk in range(TOPK):
        p = q * g_ref[kk][:, 0:C]
        qk.append(jnp.dot(p, jmat_ref[...], preferred_element_type=jnp.float32)
                  * 0.125)
    m = jnp.maximum(jnp.maximum(qk[0], qk[1]), qk[2])
    e = [jnp.exp(x - m) for x in qk]
    den = e[0] + e[1] + e[2]
    mem = (e[0] * g_ref[0][:, C:2 * C]
           + e[1] * g_ref[1][:, C:2 * C]
           + e[2] * g_ref[2][:, C:2 * C]) / den
    gate = gate_ref[...]
    comb = mem * gate + y_ref[...] * (1.0 - gate)
    out_ref[...] = jnp.dot(comb, w_ref[...], preferred_element_type=jnp.float32)


def _mem_call(qkv, g, y, gate_full, jmat, c_proj_w):
    return pl.pallas_call(
        _mem_body,
        grid=(NT,),
        in_specs=[
            pl.BlockSpec((TB, C), lambda t: (t, 0)),
            pl.BlockSpec((TOPK, TB, 2 * C), lambda t: (0, t, 0)),
            pl.BlockSpec((TB, C), lambda t: (t, 0)),
            pl.BlockSpec((1, C), lambda t: (0, 0)),
            pl.BlockSpec((C, C), lambda t: (0, 0)),
            pl.BlockSpec((C, C), lambda t: (0, 0)),
        ],
        out_specs=pl.BlockSpec((TB, C), lambda t: (t, 0)),
        out_shape=jax.ShapeDtypeStruct((T, C), jnp.float32),
    )(qkv, g, y, gate_full, jmat, c_proj_w)


def kernel(x, db_kv, c_attn_w, c_proj_w, gate_bias):
    x2 = x[0]
    db3 = db_kv[0]

    qkv, kvmem = _qkv_call(x2, c_attn_w)
    table = db3.reshape(NMEM, 2 * C)
    idxpad = _topk_call(qkv, table)
    idx_flat = idxpad[:, :TOPK].T.reshape(-1)

    g = _gather_call(table, idx_flat).reshape(TOPK, T, 2 * C)
    y = _attn_call(qkv)

    gate_full = jnp.repeat(gate_bias.reshape(H), HD)[None, :]
    seg = jnp.arange(C, dtype=jnp.int32) // HD
    jmat = (seg[:, None] == seg[None, :]).astype(jnp.float32)

    out = _mem_call(qkv, g, y, gate_full, jmat, c_proj_w)
    return out[None], kvmem[None]


# speedup vs baseline: 1.2670x; 1.2670x over previous
import functools

import jax
import jax.numpy as jnp
from jax import lax
from jax.experimental import pallas as pl
from jax.experimental.pallas import tpu as pltpu
from jax.experimental.pallas import tpu_sc as plsc

T = 2048
C = 768
H = 12
HD = 64
NMEM = 8192
TOPK = 3
TB = 256
NT = T // TB

def _qkv_body(x_ref, w_ref, qkv_ref, kv_ref):
    qkv = jnp.dot(x_ref[...], w_ref[...], preferred_element_type=jnp.float32)
    qkv_ref[...] = qkv
    kv_ref[:, 0, :] = qkv[:, C:2 * C]
    kv_ref[:, 1, :] = qkv[:, 2 * C:3 * C]


def _qkv_call(x2, c_attn_w):
    return pl.pallas_call(
        _qkv_body,
        grid=(NT,),
        in_specs=[
            pl.BlockSpec((TB, C), lambda t: (t, 0)),
            pl.BlockSpec((C, 3 * C), lambda t: (0, 0)),
        ],
        out_specs=[
            pl.BlockSpec((TB, 3 * C), lambda t: (t, 0)),
            pl.BlockSpec((TB, 2, C), lambda t: (t, 0, 0)),
        ],
        out_shape=[
            jax.ShapeDtypeStruct((T, 3 * C), jnp.float32),
            jax.ShapeDtypeStruct((T, 2, C), jnp.float32),
        ],
    )(x2, c_attn_w)


def _attn_body(q_ref, k_ref, v_ref, y_ref):
    tb = pl.program_id(1)
    row = tb * TB + lax.broadcasted_iota(jnp.int32, (TB, T), 0)
    col = lax.broadcasted_iota(jnp.int32, (TB, T), 1)
    mask = col <= row
    for h in range(2):
        q = q_ref[:, h * HD:(h + 1) * HD]
        k = k_ref[:, h * HD:(h + 1) * HD]
        v = v_ref[:, h * HD:(h + 1) * HD]
        s = lax.dot_general(q, k, (((1,), (1,)), ((), ())),
                            preferred_element_type=jnp.float32)
        s = s * 0.125
        s = jnp.where(mask, s, jnp.float32(-1e30))
        m = jnp.max(s, axis=1, keepdims=True)
        e = jnp.exp(s - m)
        den = jnp.sum(e, axis=1, keepdims=True)
        y = jnp.dot(e / den, v, preferred_element_type=jnp.float32)
        y_ref[:, h * HD:(h + 1) * HD] = y


def _attn_call(qkv):
    return pl.pallas_call(
        _attn_body,
        grid=(H // 2, NT),
        in_specs=[
            pl.BlockSpec((TB, 128), lambda hp, t: (t, hp)),
            pl.BlockSpec((T, 128), lambda hp, t: (0, 6 + hp)),
            pl.BlockSpec((T, 128), lambda hp, t: (0, 12 + hp)),
        ],
        out_specs=pl.BlockSpec((TB, 128), lambda hp, t: (t, hp)),
        out_shape=jax.ShapeDtypeStruct((T, C), jnp.float32),
    )(qkv, qkv, qkv)


def _topk_body(q_ref, mk_ref, idx_ref):
    s = lax.dot_general(q_ref[...], mk_ref[...], (((1,), (1,)), ((), ())),
                        preferred_element_type=jnp.float32)
    col = lax.broadcasted_iota(jnp.int32, (TB, NMEM), 1)
    picks = []
    for _ in range(TOPK):
        m = jnp.max(s, axis=1, keepdims=True)
        i = jnp.min(jnp.where(s == m, col, NMEM), axis=1, keepdims=True)
        picks.append(i)
        s = jnp.where(col == i, jnp.float32(-3e38), s)
    lane = lax.broadcasted_iota(jnp.int32, (TB, 128), 1)
    out = jnp.where(lane == 0, picks[0],
                    jnp.where(lane == 1, picks[1],
                              jnp.where(lane == 2, picks[2], 0)))
    idx_ref[...] = out


def _topk_call(qkv, table):
    return pl.pallas_call(
        _topk_body,
        grid=(NT,),
        in_specs=[
            pl.BlockSpec((TB, C), lambda t: (t, 0)),
            pl.BlockSpec((NMEM, C), lambda t: (0, 0)),
        ],
        out_specs=pl.BlockSpec((TB, 128), lambda t: (t, 0)),
        out_shape=jax.ShapeDtypeStruct((T, 128), jnp.int32),
    )(qkv, table)


_NROWS = TOPK * T
_NW = 32
_RPW = _NROWS // _NW
_CHUNK = 48


def _gather_call(table, idx_flat):
    mesh = plsc.VectorSubcoreMesh(core_axis_name="c", subcore_axis_name="s")

    @functools.partial(
        pl.kernel,
        mesh=mesh,
        out_type=jax.ShapeDtypeStruct((_NROWS, 2 * C), jnp.float32),
        scratch_types=[
            pltpu.VMEM((_RPW,), jnp.int32),
            pltpu.VMEM((_CHUNK, 2 * C), jnp.float32),
            pltpu.SemaphoreType.DMA,
        ],
    )
    def _gather(table_hbm, idx_hbm, out_hbm, idx_v, rows_v, sem):
        wid = lax.axis_index("s") * 2 + lax.axis_index("c")
        base = wid * _RPW
        pltpu.sync_copy(idx_hbm.at[pl.ds(base, _RPW)], idx_v)
        for ch in range(_RPW // _CHUNK):
            pltpu.async_copy(
                table_hbm.at[idx_v.at[pl.ds(ch * _CHUNK, _CHUNK)]], rows_v, sem
            ).wait()
            pltpu.sync_copy(rows_v, out_hbm.at[pl.ds(base + ch * _CHUNK, _CHUNK)])

    return _gather(table, idx_flat)


def _mem_body(q_ref, g_ref, y_ref, gate_ref, jmat_ref, w_ref, out_ref):
    q = q_ref[...]
    qk = []
    for kk in range(TOPK):
        p = q * g_ref[kk][:, 0:C]
        qk.append(jnp.dot(p, jmat_ref[...], preferred_element_type=jnp.float32)
                  * 0.125)
    m = jnp.maximum(jnp.maximum(qk[0], qk[1]), qk[2])
    e = [jnp.exp(x - m) for x in qk]
    den = e[0] + e[1] + e[2]
    mem = (e[0] * g_ref[0][:, C:2 * C]
           + e[1] * g_ref[1][:, C:2 * C]
           + e[2] * g_ref[2][:, C:2 * C]) / den
    gate = gate_ref[...]
    comb = mem * gate + y_ref[...] * (1.0 - gate)
    out_ref[...] = jnp.dot(comb, w_ref[...], preferred_element_type=jnp.float32)


def _mem_call(qkv, g, y, gate_full, jmat, c_proj_w):
    return pl.pallas_call(
        _mem_body,
        grid=(NT,),
        in_specs=[
            pl.BlockSpec((TB, C), lambda t: (t, 0)),
            pl.BlockSpec((TOPK, TB, 2 * C), lambda t: (0, t, 0)),
            pl.BlockSpec((TB, C), lambda t: (t, 0)),
            pl.BlockSpec((1, C), lambda t: (0, 0)),
            pl.BlockSpec((C, C), lambda t: (0, 0)),
            pl.BlockSpec((C, C), lambda t: (0, 0)),
        ],
        out_specs=pl.BlockSpec((TB, C), lambda t: (t, 0)),
        out_shape=jax.ShapeDtypeStruct((T, C), jnp.float32),
    )(qkv, g, y, gate_full, jmat, c_proj_w)


def kernel(x, db_kv, c_attn_w, c_proj_w, gate_bias):
    x2 = x[0]
    db3 = db_kv[0]

    qkv, kvmem = _qkv_call(x2, c_attn_w)
    table = db3.reshape(NMEM, 2 * C)
    idxpad = _topk_call(qkv, table)
    idx_flat = idxpad[:, :TOPK].T.reshape(-1)

    g = _gather_call(table, idx_flat).reshape(TOPK, T, 2 * C)
    y = _attn_call(qkv)

    gate_full = jnp.repeat(gate_bias.reshape(H), HD)[None, :]
    seg = jnp.arange(C, dtype=jnp.int32) // HD
    jmat = (seg[:, None] == seg[None, :]).astype(jnp.float32)

    out = _mem_call(qkv, g, y, gate_full, jmat, c_proj_w)
    return out[None], kvmem[None]


# speedup vs baseline: 1.3115x; 1.0351x over previous
import functools

import jax
import jax.numpy as jnp
from jax import lax
from jax.experimental import pallas as pl
from jax.experimental.pallas import tpu as pltpu
from jax.experimental.pallas import tpu_sc as plsc

T = 2048
C = 768
H = 12
HD = 64
NMEM = 8192
TOPK = 3
TB = 256
NT = T // TB

def _qkv_body(x_ref, w_ref, qkv_ref, kv_ref):
    qkv = jnp.dot(x_ref[...], w_ref[...], preferred_element_type=jnp.float32)
    qkv_ref[...] = qkv
    kv_ref[:, 0, :] = qkv[:, C:2 * C]
    kv_ref[:, 1, :] = qkv[:, 2 * C:3 * C]


def _qkv_call(x2, c_attn_w):
    return pl.pallas_call(
        _qkv_body,
        grid=(NT,),
        in_specs=[
            pl.BlockSpec((TB, C), lambda t: (t, 0)),
            pl.BlockSpec((C, 3 * C), lambda t: (0, 0)),
        ],
        out_specs=[
            pl.BlockSpec((TB, 3 * C), lambda t: (t, 0)),
            pl.BlockSpec((TB, 2, C), lambda t: (t, 0, 0)),
        ],
        out_shape=[
            jax.ShapeDtypeStruct((T, 3 * C), jnp.float32),
            jax.ShapeDtypeStruct((T, 2, C), jnp.float32),
        ],
    )(x2, c_attn_w)


def _attn_body(q_ref, k_ref, v_ref, y_ref):
    tb = pl.program_id(1)
    row = tb * TB + lax.broadcasted_iota(jnp.int32, (TB, T), 0)
    col = lax.broadcasted_iota(jnp.int32, (TB, T), 1)
    mask = col <= row
    for h in range(2):
        q = q_ref[:, h * HD:(h + 1) * HD]
        k = k_ref[:, h * HD:(h + 1) * HD]
        v = v_ref[:, h * HD:(h + 1) * HD]
        s = lax.dot_general(q, k, (((1,), (1,)), ((), ())),
                            preferred_element_type=jnp.float32)
        s = s * 0.125
        s = jnp.where(mask, s, jnp.float32(-1e30))
        m = jnp.max(s, axis=1, keepdims=True)
        e = jnp.exp(s - m)
        den = jnp.sum(e, axis=1, keepdims=True)
        y = jnp.dot(e / den, v, preferred_element_type=jnp.float32)
        y_ref[:, h * HD:(h + 1) * HD] = y


def _attn_call(qkv):
    return pl.pallas_call(
        _attn_body,
        grid=(H // 2, NT),
        in_specs=[
            pl.BlockSpec((TB, 128), lambda hp, t: (t, hp)),
            pl.BlockSpec((T, 128), lambda hp, t: (0, 6 + hp)),
            pl.BlockSpec((T, 128), lambda hp, t: (0, 12 + hp)),
        ],
        out_specs=pl.BlockSpec((TB, 128), lambda hp, t: (t, hp)),
        out_shape=jax.ShapeDtypeStruct((T, C), jnp.float32),
    )(qkv, qkv, qkv)


def _topk_body(q_ref, mk_ref, idx_ref):
    s = lax.dot_general(q_ref[...], mk_ref[...], (((1,), (1,)), ((), ())),
                        preferred_element_type=jnp.float32)
    col = lax.broadcasted_iota(jnp.int32, (TB, NMEM), 1)
    picks = []
    for _ in range(TOPK):
        i = jnp.argmax(s, axis=1, keepdims=True).astype(jnp.int32)
        picks.append(i)
        s = jnp.where(col == i, jnp.float32(-3e38), s)
    lane = lax.broadcasted_iota(jnp.int32, (TB, 128), 1)
    out = jnp.where(lane == 0, picks[0],
                    jnp.where(lane == 1, picks[1],
                              jnp.where(lane == 2, picks[2], 0)))
    idx_ref[...] = out


def _topk_call(qkv, table):
    return pl.pallas_call(
        _topk_body,
        grid=(NT,),
        in_specs=[
            pl.BlockSpec((TB, C), lambda t: (t, 0)),
            pl.BlockSpec((NMEM, C), lambda t: (0, 0)),
        ],
        out_specs=pl.BlockSpec((TB, 128), lambda t: (t, 0)),
        out_shape=jax.ShapeDtypeStruct((T, 128), jnp.int32),
    )(qkv, table)


_NROWS = TOPK * T
_NW = 32
_RPW = _NROWS // _NW
_CHUNK = 48


def _gather_call(table, idx_flat):
    mesh = plsc.VectorSubcoreMesh(core_axis_name="c", subcore_axis_name="s")

    @functools.partial(
        pl.kernel,
        mesh=mesh,
        out_type=jax.ShapeDtypeStruct((_NROWS, 2 * C), jnp.float32),
        scratch_types=[
            pltpu.VMEM((_RPW,), jnp.int32),
            pltpu.VMEM((_CHUNK, 2 * C), jnp.float32),
            pltpu.SemaphoreType.DMA,
        ],
    )
    def _gather(table_hbm, idx_hbm, out_hbm, idx_v, rows_v, sem):
        wid = lax.axis_index("s") * 2 + lax.axis_index("c")
        base = wid * _RPW
        pltpu.sync_copy(idx_hbm.at[pl.ds(base, _RPW)], idx_v)
        for ch in range(_RPW // _CHUNK):
            pltpu.async_copy(
                table_hbm.at[idx_v.at[pl.ds(ch * _CHUNK, _CHUNK)]], rows_v, sem
            ).wait()
            pltpu.sync_copy(rows_v, out_hbm.at[pl.ds(base + ch * _CHUNK, _CHUNK)])

    return _gather(table, idx_flat)


def _mem_body(q_ref, g_ref, y_ref, gate_ref, jmat_ref, w_ref, out_ref):
    q = q_ref[...]
    qk = []
    for kk in range(TOPK):
        p = q * g_ref[kk][:, 0:C]
        qk.append(jnp.dot(p, jmat_ref[...], preferred_element_type=jnp.float32)
                  * 0.125)
    m = jnp.maximum(jnp.maximum(qk[0], qk[1]), qk[2])
    e = [jnp.exp(x - m) for x in qk]
    den = e[0] + e[1] + e[2]
    mem = (e[0] * g_ref[0][:, C:2 * C]
           + e[1] * g_ref[1][:, C:2 * C]
           + e[2] * g_ref[2][:, C:2 * C]) / den
    gate = gate_ref[...]
    comb = mem * gate + y_ref[...] * (1.0 - gate)
    out_ref[...] = jnp.dot(comb, w_ref[...], preferred_element_type=jnp.float32)


def _mem_call(qkv, g, y, gate_full, jmat, c_proj_w):
    return pl.pallas_call(
        _mem_body,
        grid=(NT,),
        in_specs=[
            pl.BlockSpec((TB, C), lambda t: (t, 0)),
            pl.BlockSpec((TOPK, TB, 2 * C), lambda t: (0, t, 0)),
            pl.BlockSpec((TB, C), lambda t: (t, 0)),
            pl.BlockSpec((1, C), lambda t: (0, 0)),
            pl.BlockSpec((C, C), lambda t: (0, 0)),
            pl.BlockSpec((C, C), lambda t: (0, 0)),
        ],
        out_specs=pl.BlockSpec((TB, C), lambda t: (t, 0)),
        out_shape=jax.ShapeDtypeStruct((T, C), jnp.float32),
    )(qkv, g, y, gate_full, jmat, c_proj_w)


def kernel(x, db_kv, c_attn_w, c_proj_w, gate_bias):
    x2 = x[0]
    db3 = db_kv[0]

    qkv, kvmem = _qkv_call(x2, c_attn_w)
    table = db3.reshape(NMEM, 2 * C)
    idxpad = _topk_call(qkv, table)
    idx_flat = idxpad[:, :TOPK].T.reshape(-1)

    g = _gather_call(table, idx_flat).reshape(TOPK, T, 2 * C)
    y = _attn_call(qkv)

    gate_full = jnp.repeat(gate_bias.reshape(H), HD)[None, :]
    seg = jnp.arange(C, dtype=jnp.int32) // HD
    jmat = (seg[:, None] == seg[None, :]).astype(jnp.float32)

    out = _mem_call(qkv, g, y, gate_full, jmat, c_proj_w)
    return out[None], kvmem[None]


# speedup vs baseline: 1.3630x; 1.0393x over previous
import functools

import jax
import jax.numpy as jnp
from jax import lax
from jax.experimental import pallas as pl
from jax.experimental.pallas import tpu as pltpu
from jax.experimental.pallas import tpu_sc as plsc

T = 2048
C = 768
H = 12
HD = 64
NMEM = 8192
TOPK = 3
TB = 256
NT = T // TB

def _qkv_body(x_ref, w_ref, qkv_ref, kv_ref):
    qkv = jnp.dot(x_ref[...], w_ref[...], preferred_element_type=jnp.float32)
    qkv_ref[...] = qkv
    kv_ref[:, 0, :] = qkv[:, C:2 * C]
    kv_ref[:, 1, :] = qkv[:, 2 * C:3 * C]


def _qkv_call(x2, c_attn_w):
    return pl.pallas_call(
        _qkv_body,
        grid=(NT,),
        in_specs=[
            pl.BlockSpec((TB, C), lambda t: (t, 0)),
            pl.BlockSpec((C, 3 * C), lambda t: (0, 0)),
        ],
        out_specs=[
            pl.BlockSpec((TB, 3 * C), lambda t: (t, 0)),
            pl.BlockSpec((TB, 2, C), lambda t: (t, 0, 0)),
        ],
        out_shape=[
            jax.ShapeDtypeStruct((T, 3 * C), jnp.float32),
            jax.ShapeDtypeStruct((T, 2, C), jnp.float32),
        ],
    )(x2, c_attn_w)


TBB = 512


def _attn_body(q_ref, k_ref, v_ref, y_ref):
    tb = pl.program_id(1)
    row = tb * TBB + lax.broadcasted_iota(jnp.int32, (TBB, T), 0)
    col = lax.broadcasted_iota(jnp.int32, (TBB, T), 1)
    mask = col <= row
    for h in range(2):
        q = q_ref[:, h * HD:(h + 1) * HD]
        k = k_ref[:, h * HD:(h + 1) * HD]
        v = v_ref[:, h * HD:(h + 1) * HD]
        s = lax.dot_general(q, k, (((1,), (1,)), ((), ())),
                            preferred_element_type=jnp.float32)
        s = s * 0.125
        s = jnp.where(mask, s, jnp.float32(-1e30))
        m = jnp.max(s, axis=1, keepdims=True)
        e = jnp.exp(s - m)
        den = jnp.sum(e, axis=1, keepdims=True)
        y = jnp.dot(e / den, v, preferred_element_type=jnp.float32)
        y_ref[:, h * HD:(h + 1) * HD] = y


def _attn_call(qkv):
    return pl.pallas_call(
        _attn_body,
        grid=(H // 2, T // TBB),
        in_specs=[
            pl.BlockSpec((TBB, 128), lambda hp, t: (t, hp)),
            pl.BlockSpec((T, 128), lambda hp, t: (0, 6 + hp)),
            pl.BlockSpec((T, 128), lambda hp, t: (0, 12 + hp)),
        ],
        out_specs=pl.BlockSpec((TBB, 128), lambda hp, t: (t, hp)),
        out_shape=jax.ShapeDtypeStruct((T, C), jnp.float32),
    )(qkv, qkv, qkv)


def _topk_body(q_ref, mk_ref, idx_ref):
    s = lax.dot_general(q_ref[...], mk_ref[...], (((1,), (1,)), ((), ())),
                        preferred_element_type=jnp.float32)
    col = lax.broadcasted_iota(jnp.int32, (TB, NMEM), 1)
    picks = []
    for _ in range(TOPK):
        i = jnp.argmax(s, axis=1, keepdims=True).astype(jnp.int32)
        picks.append(i)
        s = jnp.where(col == i, jnp.float32(-3e38), s)
    lane = lax.broadcasted_iota(jnp.int32, (TB, 128), 1)
    out = jnp.where(lane == 0, picks[0],
                    jnp.where(lane == 1, picks[1],
                              jnp.where(lane == 2, picks[2], 0)))
    idx_ref[...] = out


def _topk_call(qkv, table):
    return pl.pallas_call(
        _topk_body,
        grid=(NT,),
        in_specs=[
            pl.BlockSpec((TB, C), lambda t: (t, 0)),
            pl.BlockSpec((NMEM, C), lambda t: (0, 0)),
        ],
        out_specs=pl.BlockSpec((TB, 128), lambda t: (t, 0)),
        out_shape=jax.ShapeDtypeStruct((T, 128), jnp.int32),
    )(qkv, table)


_NROWS = TOPK * T
_NW = 32
_RPW = _NROWS // _NW
_CHUNK = 48


def _gather_call(table, idx_flat):
    mesh = plsc.VectorSubcoreMesh(core_axis_name="c", subcore_axis_name="s")

    @functools.partial(
        pl.kernel,
        mesh=mesh,
        out_type=jax.ShapeDtypeStruct((_NROWS, 2 * C), jnp.float32),
        scratch_types=[
            pltpu.VMEM((_RPW,), jnp.int32),
            pltpu.VMEM((_CHUNK, 2 * C), jnp.float32),
            pltpu.SemaphoreType.DMA,
        ],
    )
    def _gather(table_hbm, idx_hbm, out_hbm, idx_v, rows_v, sem):
        wid = lax.axis_index("s") * 2 + lax.axis_index("c")
        base = wid * _RPW
        pltpu.sync_copy(idx_hbm.at[pl.ds(base, _RPW)], idx_v)
        for ch in range(_RPW // _CHUNK):
            pltpu.async_copy(
                table_hbm.at[idx_v.at[pl.ds(ch * _CHUNK, _CHUNK)]], rows_v, sem
            ).wait()
            pltpu.sync_copy(rows_v, out_hbm.at[pl.ds(base + ch * _CHUNK, _CHUNK)])

    return _gather(table, idx_flat)


def _mem_body(q_ref, g_ref, y_ref, gate_ref, jmat_ref, w_ref, out_ref):
    q = q_ref[...]
    qk = []
    for kk in range(TOPK):
        p = q * g_ref[kk][:, 0:C]
        qk.append(jnp.dot(p, jmat_ref[...], preferred_element_type=jnp.float32)
                  * 0.125)
    m = jnp.maximum(jnp.maximum(qk[0], qk[1]), qk[2])
    e = [jnp.exp(x - m) for x in qk]
    den = e[0] + e[1] + e[2]
    mem = (e[0] * g_ref[0][:, C:2 * C]
           + e[1] * g_ref[1][:, C:2 * C]
           + e[2] * g_ref[2][:, C:2 * C]) / den
    gate = gate_ref[...]
    comb = mem * gate + y_ref[...] * (1.0 - gate)
    out_ref[...] = jnp.dot(comb, w_ref[...], preferred_element_type=jnp.float32)


def _mem_call(qkv, g, y, gate_full, jmat, c_proj_w):
    return pl.pallas_call(
        _mem_body,
        grid=(NT,),
        in_specs=[
            pl.BlockSpec((TB, C), lambda t: (t, 0)),
            pl.BlockSpec((TOPK, TB, 2 * C), lambda t: (0, t, 0)),
            pl.BlockSpec((TB, C), lambda t: (t, 0)),
            pl.BlockSpec((1, C), lambda t: (0, 0)),
            pl.BlockSpec((C, C), lambda t: (0, 0)),
            pl.BlockSpec((C, C), lambda t: (0, 0)),
        ],
        out_specs=pl.BlockSpec((TB, C), lambda t: (t, 0)),
        out_shape=jax.ShapeDtypeStruct((T, C), jnp.float32),
    )(qkv, g, y, gate_full, jmat, c_proj_w)


def kernel(x, db_kv, c_attn_w, c_proj_w, gate_bias):
    x2 = x[0]
    db3 = db_kv[0]

    qkv, kvmem = _qkv_call(x2, c_attn_w)
    table = db3.reshape(NMEM, 2 * C)
    idxpad = _topk_call(qkv, table)
    idx_flat = idxpad[:, :TOPK].T.reshape(-1)

    g = _gather_call(table, idx_flat).reshape(TOPK, T, 2 * C)
    y = _attn_call(qkv)

    gate_full = jnp.repeat(gate_bias.reshape(H), HD)[None, :]
    seg = jnp.arange(C, dtype=jnp.int32) // HD
    jmat = (seg[:, None] == seg[None, :]).astype(jnp.float32)

    out = _mem_call(qkv, g, y, gate_full, jmat, c_proj_w)
    return out[None], kvmem[None]


# speedup vs baseline: 1.3898x; 1.0196x over previous
import functools

import jax
import jax.numpy as jnp
from jax import lax
from jax.experimental import pallas as pl
from jax.experimental.pallas import tpu as pltpu
from jax.experimental.pallas import tpu_sc as plsc

T = 2048
C = 768
H = 12
HD = 64
NMEM = 8192
TOPK = 3
TB = 256
NT = T // TB

def _qkv_body(x_ref, w_ref, qkv_ref, kv_ref):
    qkv = jnp.dot(x_ref[...], w_ref[...], preferred_element_type=jnp.float32)
    qkv_ref[...] = qkv
    kv_ref[:, 0, :] = qkv[:, C:2 * C]
    kv_ref[:, 1, :] = qkv[:, 2 * C:3 * C]


def _qkv_call(x2, c_attn_w):
    return pl.pallas_call(
        _qkv_body,
        grid=(NT,),
        in_specs=[
            pl.BlockSpec((TB, C), lambda t: (t, 0)),
            pl.BlockSpec((C, 3 * C), lambda t: (0, 0)),
        ],
        out_specs=[
            pl.BlockSpec((TB, 3 * C), lambda t: (t, 0)),
            pl.BlockSpec((TB, 2, C), lambda t: (t, 0, 0)),
        ],
        out_shape=[
            jax.ShapeDtypeStruct((T, 3 * C), jnp.float32),
            jax.ShapeDtypeStruct((T, 2, C), jnp.float32),
        ],
    )(x2, c_attn_w)


TBB = 1024


def _attn_body(q_ref, k_ref, v_ref, y_ref):
    tb = pl.program_id(1)
    row = tb * TBB + lax.broadcasted_iota(jnp.int32, (TBB, T), 0)
    col = lax.broadcasted_iota(jnp.int32, (TBB, T), 1)
    mask = col <= row
    for h in range(2):
        q = q_ref[:, h * HD:(h + 1) * HD]
        k = k_ref[:, h * HD:(h + 1) * HD]
        v = v_ref[:, h * HD:(h + 1) * HD]
        s = lax.dot_general(q, k, (((1,), (1,)), ((), ())),
                            preferred_element_type=jnp.float32)
        s = s * 0.125
        s = jnp.where(mask, s, jnp.float32(-1e30))
        m = jnp.max(s, axis=1, keepdims=True)
        e = jnp.exp(s - m)
        den = jnp.sum(e, axis=1, keepdims=True)
        y = jnp.dot(e / den, v, preferred_element_type=jnp.float32)
        y_ref[:, h * HD:(h + 1) * HD] = y


def _attn_call(qkv):
    return pl.pallas_call(
        _attn_body,
        grid=(H // 2, T // TBB),
        in_specs=[
            pl.BlockSpec((TBB, 128), lambda hp, t: (t, hp)),
            pl.BlockSpec((T, 128), lambda hp, t: (0, 6 + hp)),
            pl.BlockSpec((T, 128), lambda hp, t: (0, 12 + hp)),
        ],
        out_specs=pl.BlockSpec((TBB, 128), lambda hp, t: (t, hp)),
        out_shape=jax.ShapeDtypeStruct((T, C), jnp.float32),
    )(qkv, qkv, qkv)


def _topk_body(q_ref, mk_ref, idx_ref):
    s = lax.dot_general(q_ref[...], mk_ref[...], (((1,), (1,)), ((), ())),
                        preferred_element_type=jnp.float32)
    col = lax.broadcasted_iota(jnp.int32, (TB, NMEM), 1)
    picks = []
    for _ in range(TOPK):
        i = jnp.argmax(s, axis=1, keepdims=True).astype(jnp.int32)
        picks.append(i)
        s = jnp.where(col == i, jnp.float32(-3e38), s)
    lane = lax.broadcasted_iota(jnp.int32, (TB, 128), 1)
    out = jnp.where(lane == 0, picks[0],
                    jnp.where(lane == 1, picks[1],
                              jnp.where(lane == 2, picks[2], 0)))
    idx_ref[...] = out


def _topk_call(qkv, table):
    return pl.pallas_call(
        _topk_body,
        grid=(NT,),
        in_specs=[
            pl.BlockSpec((TB, C), lambda t: (t, 0)),
            pl.BlockSpec((NMEM, C), lambda t: (0, 0)),
        ],
        out_specs=pl.BlockSpec((TB, 128), lambda t: (t, 0)),
        out_shape=jax.ShapeDtypeStruct((T, 128), jnp.int32),
    )(qkv, table)


_NROWS = TOPK * T
_NW = 32
_RPW = _NROWS // _NW
_CHUNK = 48


def _gather_call(table, idx_flat):
    mesh = plsc.VectorSubcoreMesh(core_axis_name="c", subcore_axis_name="s")

    @functools.partial(
        pl.kernel,
        mesh=mesh,
        out_type=jax.ShapeDtypeStruct((_NROWS, 2 * C), jnp.float32),
        scratch_types=[
            pltpu.VMEM((_RPW,), jnp.int32),
            pltpu.VMEM((_CHUNK, 2 * C), jnp.float32),
            pltpu.SemaphoreType.DMA,
        ],
    )
    def _gather(table_hbm, idx_hbm, out_hbm, idx_v, rows_v, sem):
        wid = lax.axis_index("s") * 2 + lax.axis_index("c")
        base = wid * _RPW
        pltpu.sync_copy(idx_hbm.at[pl.ds(base, _RPW)], idx_v)
        for ch in range(_RPW // _CHUNK):
            pltpu.async_copy(
                table_hbm.at[idx_v.at[pl.ds(ch * _CHUNK, _CHUNK)]], rows_v, sem
            ).wait()
            pltpu.sync_copy(rows_v, out_hbm.at[pl.ds(base + ch * _CHUNK, _CHUNK)])

    return _gather(table, idx_flat)


def _mem_body(q_ref, g_ref, y_ref, gate_ref, jmat_ref, w_ref, out_ref):
    q = q_ref[...]
    qk = []
    for kk in range(TOPK):
        p = q * g_ref[kk][:, 0:C]
        qk.append(jnp.dot(p, jmat_ref[...], preferred_element_type=jnp.float32)
                  * 0.125)
    m = jnp.maximum(jnp.maximum(qk[0], qk[1]), qk[2])
    e = [jnp.exp(x - m) for x in qk]
    den = e[0] + e[1] + e[2]
    mem = (e[0] * g_ref[0][:, C:2 * C]
           + e[1] * g_ref[1][:, C:2 * C]
           + e[2] * g_ref[2][:, C:2 * C]) / den
    gate = gate_ref[...]
    comb = mem * gate + y_ref[...] * (1.0 - gate)
    out_ref[...] = jnp.dot(comb, w_ref[...], preferred_element_type=jnp.float32)


def _mem_call(qkv, g, y, gate_full, jmat, c_proj_w):
    return pl.pallas_call(
        _mem_body,
        grid=(NT,),
        in_specs=[
            pl.BlockSpec((TB, C), lambda t: (t, 0)),
            pl.BlockSpec((TOPK, TB, 2 * C), lambda t: (0, t, 0)),
            pl.BlockSpec((TB, C), lambda t: (t, 0)),
            pl.BlockSpec((1, C), lambda t: (0, 0)),
            pl.BlockSpec((C, C), lambda t: (0, 0)),
            pl.BlockSpec((C, C), lambda t: (0, 0)),
        ],
        out_specs=pl.BlockSpec((TB, C), lambda t: (t, 0)),
        out_shape=jax.ShapeDtypeStruct((T, C), jnp.float32),
    )(qkv, g, y, gate_full, jmat, c_proj_w)


def kernel(x, db_kv, c_attn_w, c_proj_w, gate_bias):
    x2 = x[0]
    db3 = db_kv[0]

    qkv, kvmem = _qkv_call(x2, c_attn_w)
    table = db3.reshape(NMEM, 2 * C)
    idxpad = _topk_call(qkv, table)
    idx_flat = idxpad[:, :TOPK].T.reshape(-1)

    g = _gather_call(table, idx_flat).reshape(TOPK, T, 2 * C)
    y = _attn_call(qkv)

    gate_full = jnp.repeat(gate_bias.reshape(H), HD)[None, :]
    seg = jnp.arange(C, dtype=jnp.int32) // HD
    jmat = (seg[:, None] == seg[None, :]).astype(jnp.float32)

    out = _mem_call(qkv, g, y, gate_full, jmat, c_proj_w)
    return out[None], kvmem[None]


# speedup vs baseline: 1.4107x; 1.0150x over previous
import functools

import jax
import jax.numpy as jnp
from jax import lax
from jax.experimental import pallas as pl
from jax.experimental.pallas import tpu as pltpu
from jax.experimental.pallas import tpu_sc as plsc

T = 2048
C = 768
H = 12
HD = 64
NMEM = 8192
TOPK = 3
TB = 512
NT = T // TB

def _qkv_body(x_ref, w_ref, qkv_ref, kv_ref):
    qkv = jnp.dot(x_ref[...], w_ref[...], preferred_element_type=jnp.float32)
    qkv_ref[...] = qkv
    kv_ref[:, 0, :] = qkv[:, C:2 * C]
    kv_ref[:, 1, :] = qkv[:, 2 * C:3 * C]


def _qkv_call(x2, c_attn_w):
    return pl.pallas_call(
        _qkv_body,
        grid=(NT,),
        in_specs=[
            pl.BlockSpec((TB, C), lambda t: (t, 0)),
            pl.BlockSpec((C, 3 * C), lambda t: (0, 0)),
        ],
        out_specs=[
            pl.BlockSpec((TB, 3 * C), lambda t: (t, 0)),
            pl.BlockSpec((TB, 2, C), lambda t: (t, 0, 0)),
        ],
        out_shape=[
            jax.ShapeDtypeStruct((T, 3 * C), jnp.float32),
            jax.ShapeDtypeStruct((T, 2, C), jnp.float32),
        ],
    )(x2, c_attn_w)


TBB = 1024


def _attn_body(q_ref, k_ref, v_ref, y_ref):
    tb = pl.program_id(1)
    row = tb * TBB + lax.broadcasted_iota(jnp.int32, (TBB, T), 0)
    col = lax.broadcasted_iota(jnp.int32, (TBB, T), 1)
    mask = col <= row
    for h in range(2):
        q = q_ref[:, h * HD:(h + 1) * HD]
        k = k_ref[:, h * HD:(h + 1) * HD]
        v = v_ref[:, h * HD:(h + 1) * HD]
        s = lax.dot_general(q, k, (((1,), (1,)), ((), ())),
                            preferred_element_type=jnp.float32)
        s = s * 0.125
        s = jnp.where(mask, s, jnp.float32(-1e30))
        m = jnp.max(s, axis=1, keepdims=True)
        e = jnp.exp(s - m)
        den = jnp.sum(e, axis=1, keepdims=True)
        y = jnp.dot(e / den, v, preferred_element_type=jnp.float32)
        y_ref[:, h * HD:(h + 1) * HD] = y


def _attn_call(qkv):
    return pl.pallas_call(
        _attn_body,
        grid=(H // 2, T // TBB),
        in_specs=[
            pl.BlockSpec((TBB, 128), lambda hp, t: (t, hp)),
            pl.BlockSpec((T, 128), lambda hp, t: (0, 6 + hp)),
            pl.BlockSpec((T, 128), lambda hp, t: (0, 12 + hp)),
        ],
        out_specs=pl.BlockSpec((TBB, 128), lambda hp, t: (t, hp)),
        out_shape=jax.ShapeDtypeStruct((T, C), jnp.float32),
    )(qkv, qkv, qkv)


def _topk_body(q_ref, mk_ref, idx_ref):
    s = lax.dot_general(q_ref[...], mk_ref[...], (((1,), (1,)), ((), ())),
                        preferred_element_type=jnp.float32)
    col = lax.broadcasted_iota(jnp.int32, (TB, NMEM), 1)
    picks = []
    for _ in range(TOPK):
        i = jnp.argmax(s, axis=1, keepdims=True).astype(jnp.int32)
        picks.append(i)
        s = jnp.where(col == i, jnp.float32(-3e38), s)
    lane = lax.broadcasted_iota(jnp.int32, (TB, 128), 1)
    out = jnp.where(lane == 0, picks[0],
                    jnp.where(lane == 1, picks[1],
                              jnp.where(lane == 2, picks[2], 0)))
    idx_ref[...] = out


def _topk_call(qkv, table):
    return pl.pallas_call(
        _topk_body,
        grid=(NT,),
        in_specs=[
            pl.BlockSpec((TB, C), lambda t: (t, 0)),
            pl.BlockSpec((NMEM, C), lambda t: (0, 0)),
        ],
        out_specs=pl.BlockSpec((TB, 128), lambda t: (t, 0)),
        out_shape=jax.ShapeDtypeStruct((T, 128), jnp.int32),
    )(qkv, table)


_NROWS = TOPK * T
_NW = 32
_RPW = _NROWS // _NW
_CHUNK = 48


def _gather_call(table, idx_flat):
    mesh = plsc.VectorSubcoreMesh(core_axis_name="c", subcore_axis_name="s")

    @functools.partial(
        pl.kernel,
        mesh=mesh,
        out_type=jax.ShapeDtypeStruct((_NROWS, 2 * C), jnp.float32),
        scratch_types=[
            pltpu.VMEM((_RPW,), jnp.int32),
            pltpu.VMEM((_CHUNK, 2 * C), jnp.float32),
            pltpu.SemaphoreType.DMA,
        ],
    )
    def _gather(table_hbm, idx_hbm, out_hbm, idx_v, rows_v, sem):
        wid = lax.axis_index("s") * 2 + lax.axis_index("c")
        base = wid * _RPW
        pltpu.sync_copy(idx_hbm.at[pl.ds(base, _RPW)], idx_v)
        for ch in range(_RPW // _CHUNK):
            pltpu.async_copy(
                table_hbm.at[idx_v.at[pl.ds(ch * _CHUNK, _CHUNK)]], rows_v, sem
            ).wait()
            pltpu.sync_copy(rows_v, out_hbm.at[pl.ds(base + ch * _CHUNK, _CHUNK)])

    return _gather(table, idx_flat)


def _mem_body(q_ref, g_ref, y_ref, gate_ref, jmat_ref, w_ref, out_ref):
    q = q_ref[...]
    qk = []
    for kk in range(TOPK):
        p = q * g_ref[kk][:, 0:C]
        qk.append(jnp.dot(p, jmat_ref[...], preferred_element_type=jnp.float32)
                  * 0.125)
    m = jnp.maximum(jnp.maximum(qk[0], qk[1]), qk[2])
    e = [jnp.exp(x - m) for x in qk]
    den = e[0] + e[1] + e[2]
    mem = (e[0] * g_ref[0][:, C:2 * C]
           + e[1] * g_ref[1][:, C:2 * C]
           + e[2] * g_ref[2][:, C:2 * C]) / den
    gate = gate_ref[...]
    comb = mem * gate + y_ref[...] * (1.0 - gate)
    out_ref[...] = jnp.dot(comb, w_ref[...], preferred_element_type=jnp.float32)


def _mem_call(qkv, g, y, gate_full, jmat, c_proj_w):
    return pl.pallas_call(
        _mem_body,
        grid=(NT,),
        in_specs=[
            pl.BlockSpec((TB, C), lambda t: (t, 0)),
            pl.BlockSpec((TOPK, TB, 2 * C), lambda t: (0, t, 0)),
            pl.BlockSpec((TB, C), lambda t: (t, 0)),
            pl.BlockSpec((1, C), lambda t: (0, 0)),
            pl.BlockSpec((C, C), lambda t: (0, 0)),
            pl.BlockSpec((C, C), lambda t: (0, 0)),
        ],
        out_specs=pl.BlockSpec((TB, C), lambda t: (t, 0)),
        out_shape=jax.ShapeDtypeStruct((T, C), jnp.float32),
    )(qkv, g, y, gate_full, jmat, c_proj_w)


def kernel(x, db_kv, c_attn_w, c_proj_w, gate_bias):
    x2 = x[0]
    db3 = db_kv[0]

    qkv, kvmem = _qkv_call(x2, c_attn_w)
    table = db3.reshape(NMEM, 2 * C)
    idxpad = _topk_call(qkv, table)
    idx_flat = idxpad[:, :TOPK].T.reshape(-1)

    g = _gather_call(table, idx_flat).reshape(TOPK, T, 2 * C)
    y = _attn_call(qkv)

    gate_full = jnp.repeat(gate_bias.reshape(H), HD)[None, :]
    seg = jnp.arange(C, dtype=jnp.int32) // HD
    jmat = (seg[:, None] == seg[None, :]).astype(jnp.float32)

    out = _mem_call(qkv, g, y, gate_full, jmat, c_proj_w)
    return out[None], kvmem[None]


# speedup vs baseline: 1.6839x; 1.1937x over previous
import functools

import jax
import jax.numpy as jnp
from jax import lax
from jax.experimental import pallas as pl
from jax.experimental.pallas import tpu as pltpu
from jax.experimental.pallas import tpu_sc as plsc

T = 2048
C = 768
H = 12
HD = 64
NMEM = 8192
TOPK = 3
TB = 512
NT = T // TB

def _qkv_body(x_ref, w_ref, qkv_ref, kv_ref):
    qkv = jnp.dot(x_ref[...], w_ref[...], preferred_element_type=jnp.float32)
    qkv_ref[...] = qkv
    kv_ref[:, 0, :] = qkv[:, C:2 * C]
    kv_ref[:, 1, :] = qkv[:, 2 * C:3 * C]


def _qkv_call(x2, c_attn_w):
    return pl.pallas_call(
        _qkv_body,
        grid=(NT,),
        in_specs=[
            pl.BlockSpec((TB, C), lambda t: (t, 0)),
            pl.BlockSpec((C, 3 * C), lambda t: (0, 0)),
        ],
        out_specs=[
            pl.BlockSpec((TB, 3 * C), lambda t: (t, 0)),
            pl.BlockSpec((TB, 2, C), lambda t: (t, 0, 0)),
        ],
        out_shape=[
            jax.ShapeDtypeStruct((T, 3 * C), jnp.float32),
            jax.ShapeDtypeStruct((T, 2, C), jnp.float32),
        ],
    )(x2, c_attn_w)


TBB = 1024


def _attn_part(qkv, rblk, kext):
    def body(q_ref, k_ref, v_ref, y_ref):
        row = rblk * TBB + lax.broadcasted_iota(jnp.int32, (TBB, kext), 0)
        col = lax.broadcasted_iota(jnp.int32, (TBB, kext), 1)
        mask = col <= row
        for h in range(2):
            q = q_ref[:, h * HD:(h + 1) * HD]
            k = k_ref[:, h * HD:(h + 1) * HD]
            v = v_ref[:, h * HD:(h + 1) * HD]
            s = lax.dot_general(q, k, (((1,), (1,)), ((), ())),
                                preferred_element_type=jnp.float32)
            s = s * 0.125
            s = jnp.where(mask, s, jnp.float32(-1e30))
            m = jnp.max(s, axis=1, keepdims=True)
            e = jnp.exp(s - m)
            den = jnp.sum(e, axis=1, keepdims=True)
            y = jnp.dot(e, v, preferred_element_type=jnp.float32) / den
            y_ref[:, h * HD:(h + 1) * HD] = y

    return pl.pallas_call(
        body,
        grid=(H // 2,),
        in_specs=[
            pl.BlockSpec((TBB, 128), lambda hp: (rblk, hp)),
            pl.BlockSpec((kext, 128), lambda hp: (0, 6 + hp)),
            pl.BlockSpec((kext, 128), lambda hp: (0, 12 + hp)),
        ],
        out_specs=pl.BlockSpec((TBB, 128), lambda hp: (0, hp)),
        out_shape=jax.ShapeDtypeStruct((TBB, C), jnp.float32),
    )(qkv, qkv, qkv)


def _attn_call(qkv):
    y0 = _attn_part(qkv, 0, T // 2)
    y1 = _attn_part(qkv, 1, T)
    return jnp.concatenate([y0, y1], axis=0)


def _topk_body(q_ref, mk_ref, idx_ref):
    s = lax.dot_general(q_ref[...], mk_ref[...], (((1,), (1,)), ((), ())),
                        preferred_element_type=jnp.float32)
    col = lax.broadcasted_iota(jnp.int32, (TB, NMEM), 1)
    picks = []
    for _ in range(TOPK):
        i = jnp.argmax(s, axis=1, keepdims=True).astype(jnp.int32)
        picks.append(i)
        s = jnp.where(col == i, jnp.float32(-3e38), s)
    lane = lax.broadcasted_iota(jnp.int32, (TB, 128), 1)
    out = jnp.where(lane == 0, picks[0],
                    jnp.where(lane == 1, picks[1],
                              jnp.where(lane == 2, picks[2], 0)))
    idx_ref[...] = out


def _topk_call(qkv, table):
    return pl.pallas_call(
        _topk_body,
        grid=(NT,),
        in_specs=[
            pl.BlockSpec((TB, C), lambda t: (t, 0)),
            pl.BlockSpec((NMEM, C), lambda t: (0, 0)),
        ],
        out_specs=pl.BlockSpec((TB, 128), lambda t: (t, 0)),
        out_shape=jax.ShapeDtypeStruct((T, 128), jnp.int32),
    )(qkv, table)


_NROWS = TOPK * T
_NW = 32
_RPW = _NROWS // _NW
_CHUNK = 48


def _gather_call(table, idx_flat):
    mesh = plsc.VectorSubcoreMesh(core_axis_name="c", subcore_axis_name="s")

    @functools.partial(
        pl.kernel,
        mesh=mesh,
        out_type=jax.ShapeDtypeStruct((_NROWS, 2 * C), jnp.float32),
        scratch_types=[
            pltpu.VMEM((_RPW,), jnp.int32),
            pltpu.VMEM((_CHUNK, 2 * C), jnp.float32),
            pltpu.SemaphoreType.DMA,
        ],
    )
    def _gather(table_hbm, idx_hbm, out_hbm, idx_v, rows_v, sem):
        wid = lax.axis_index("s") * 2 + lax.axis_index("c")
        base = wid * _RPW
        pltpu.sync_copy(idx_hbm.at[pl.ds(base, _RPW)], idx_v)
        for ch in range(_RPW // _CHUNK):
            pltpu.async_copy(
                table_hbm.at[idx_v.at[pl.ds(ch * _CHUNK, _CHUNK)]], rows_v, sem
            ).wait()
            pltpu.sync_copy(rows_v, out_hbm.at[pl.ds(base + ch * _CHUNK, _CHUNK)])

    return _gather(table, idx_flat)


def _mem_body(q_ref, g_ref, y_ref, gate_ref, jmat_ref, w_ref, out_ref):
    q = q_ref[...]
    qk = []
    for kk in range(TOPK):
        p = q * g_ref[kk][:, 0:C]
        qk.append(jnp.dot(p, jmat_ref[...], preferred_element_type=jnp.float32)
                  * 0.125)
    m = jnp.maximum(jnp.maximum(qk[0], qk[1]), qk[2])
    e = [jnp.exp(x - m) for x in qk]
    den = e[0] + e[1] + e[2]
    mem = (e[0] * g_ref[0][:, C:2 * C]
           + e[1] * g_ref[1][:, C:2 * C]
           + e[2] * g_ref[2][:, C:2 * C]) / den
    gate = gate_ref[...]
    comb = mem * gate + y_ref[...] * (1.0 - gate)
    out_ref[...] = jnp.dot(comb, w_ref[...], preferred_element_type=jnp.float32)


def _mem_call(qkv, g, y, gate_full, jmat, c_proj_w):
    return pl.pallas_call(
        _mem_body,
        grid=(NT,),
        in_specs=[
            pl.BlockSpec((TB, C), lambda t: (t, 0)),
            pl.BlockSpec((TOPK, TB, 2 * C), lambda t: (0, t, 0)),
            pl.BlockSpec((TB, C), lambda t: (t, 0)),
            pl.BlockSpec((1, C), lambda t: (0, 0)),
            pl.BlockSpec((C, C), lambda t: (0, 0)),
            pl.BlockSpec((C, C), lambda t: (0, 0)),
        ],
        out_specs=pl.BlockSpec((TB, C), lambda t: (t, 0)),
        out_shape=jax.ShapeDtypeStruct((T, C), jnp.float32),
    )(qkv, g, y, gate_full, jmat, c_proj_w)


def kernel(x, db_kv, c_attn_w, c_proj_w, gate_bias):
    x2 = x[0]
    db3 = db_kv[0]

    qkv, kvmem = _qkv_call(x2, c_attn_w)
    table = db3.reshape(NMEM, 2 * C)
    idxpad = _topk_call(qkv, table)
    idx_flat = idxpad[:, :TOPK].T.reshape(-1)

    g = _gather_call(table, idx_flat).reshape(TOPK, T, 2 * C)
    y = _attn_call(qkv)

    gate_full = jnp.repeat(gate_bias.reshape(H), HD)[None, :]
    seg = jnp.arange(C, dtype=jnp.int32) // HD
    jmat = (seg[:, None] == seg[None, :]).astype(jnp.float32)

    out = _mem_call(qkv, g, y, gate_full, jmat, c_proj_w)
    return out[None], kvmem[None]
